# baseline (speedup 1.0000x reference)
import jax
import jax.numpy as jnp
from jax.experimental import pallas as pl
from jax.experimental.pallas import tpu as pltpu


_CP = pltpu.CompilerParams(
    dimension_semantics=("parallel",),
    vmem_limit_bytes=64 * 1024 * 1024,
)


def _pick_b(n, h, w):
    pix = h * w
    want = 8 if pix >= 256 else (16 if pix >= 64 else 32)
    while want > 1 and n % want:
        want //= 2
    return max(want, 1)


def _conv_body(offs, mrows, relu, mode, nb):
    def _accum(x_ref, w_ref):
        acc = None
        for t, off in enumerate(offs):
            xs = x_ref[:, off:off + mrows, :].reshape(nb * mrows, x_ref.shape[-1])
            d = jnp.dot(xs, w_ref[t], preferred_element_type=jnp.float32)
            acc = d if acc is None else acc + d
        return acc

    if mode == "plain":
        def _body(x_ref, w_ref, sc_ref, b_ref, o_ref):
            y = _accum(x_ref, w_ref) * sc_ref[...] + b_ref[...]
            y = jnp.maximum(y, 0.0) if relu else y
            o_ref[...] = y.reshape(o_ref.shape).astype(o_ref.dtype)
    elif mode == "res":
        def _body(x_ref, w_ref, sc_ref, b_ref, r_ref, o_ref):
            y = _accum(x_ref, w_ref) * sc_ref[...] + b_ref[...]
            y = y + r_ref[...].reshape(y.shape).astype(jnp.float32)
            y = jnp.maximum(y, 0.0) if relu else y
            o_ref[...] = y.reshape(o_ref.shape).astype(o_ref.dtype)
    else:
        def _body(x_ref, w_ref, sc_ref, b_ref,
                  s_ref, ws_ref, ssc_ref, sb_ref, o_ref):
            y = _accum(x_ref, w_ref) * sc_ref[...] + b_ref[...]
            ss = s_ref[...].reshape(nb * mrows, s_ref.shape[-1])
            proj = jnp.dot(ss, ws_ref[...], preferred_element_type=jnp.float32)
            y = y + proj * ssc_ref[...] + sb_ref[...]
            y = jnp.maximum(y, 0.0) if relu else y
            o_ref[...] = y.reshape(o_ref.shape).astype(o_ref.dtype)
    return _body


def _rdb_body(num_layers, offs, mrows, row0, wpad, wvalid, nb):
    ntaps = len(offs)

    def _body(*refs):
        x_ref = refs[0]
        lw = refs[1:1 + 2 * num_layers]
        wlff_ref = refs[1 + 2 * num_layers]
        blff_ref = refs[2 + 2 * num_layers]
        o_ref = refs[3 + 2 * num_layers]
        canvas = refs[4 + 2 * num_layers]

        canvas[...] = jnp.zeros_like(canvas)
        canvas[:, 0] = x_ref[...]

        col = jax.lax.broadcasted_iota(jnp.int32, (nb * mrows, 1), 0) % wpad
        valid = col < wvalid

        for j in range(num_layers):
            w_ref, b_ref = lw[2 * j], lw[2 * j + 1]
            acc = None
            for c in range(j + 1):
                for t, off in enumerate(offs):
                    xs = canvas[:, c, off:off + mrows, :].reshape(nb * mrows, 64)
                    d = jnp.dot(xs, w_ref[c * ntaps + t],
                                preferred_element_type=jnp.float32)
                    acc = d if acc is None else acc + d
            y = jnp.maximum(acc + b_ref[...], 0.0)
            y = jnp.where(valid, y, 0.0)
            canvas[:, j + 1, row0:row0 + mrows, :] = (
                y.reshape(nb, mrows, 64).astype(canvas.dtype))

        acc = None
        for c in range(num_layers + 1):
            xs = canvas[:, c, row0:row0 + mrows, :].reshape(nb * mrows, 64)
            d = jnp.dot(xs, wlff_ref[c], preferred_element_type=jnp.float32)
            acc = d if acc is None else acc + d
        res = canvas[:, 0, row0:row0 + mrows, :].reshape(nb * mrows, 64)
        out = acc + blff_ref[...] + res.astype(jnp.float32)
        o_ref[...] = out.reshape(nb, mrows, 64).astype(o_ref.dtype)

    return _body


def _to_canvas(x, pt, pb, pleft, pr, min_rows):
    n, h, w, c = x.shape
    xc = jnp.pad(x.astype(jnp.bfloat16), ((0, 0), (pt, pb), (pleft, pr), (0, 0)))
    hp, wp = h + pt + pb, w + pleft + pr
    xc = xc.reshape(n, hp * wp, c)
    if min_rows > hp * wp:
        xc = jnp.pad(xc, ((0, 0), (0, min_rows - hp * wp), (0, 0)))
    return xc


def _conv(x, pk, *, pads=None, relu=False, residual=None, shortcut=None):
    n, h, w, cin = x.shape
    kh, kw, cout, cp = pk["kh"], pk["kw"], pk["cout"], pk["cp"]
    if pads is None:
        pads = (kh // 2, kh // 2, kw // 2, kw // 2)
    pt, pb, pleft, pr = pads
    hp, wp = h + pt + pb, w + pleft + pr
    ho, wo = hp - kh + 1, wp - kw + 1
    mrows = ho * wp
    min_rows = hp * wp + (kw - 1)
    offs = [dy * wp + dx for dy in range(kh) for dx in range(kw)]
    nb = _pick_b(n, h, w)

    xc = _to_canvas(x, pt, pb, pleft, pr, min_rows)

    inputs = [xc, pk["wt"], pk["sc"], pk["b"]]
    in_specs = [
        pl.BlockSpec((nb, min_rows, cin), lambda i: (i, 0, 0)),
        pl.BlockSpec(pk["wt"].shape, lambda i: (0, 0, 0)),
        pl.BlockSpec((1, cp), lambda i: (0, 0)),
        pl.BlockSpec((1, cp), lambda i: (0, 0)),
    ]

    if residual is not None:
        mode = "res"
        r = jnp.pad(residual.astype(jnp.bfloat16),
                    ((0, 0), (0, 0), (0, wp - wo), (0, cp - cout)))
        inputs.append(r.reshape(n, mrows, cp))
        in_specs.append(pl.BlockSpec((nb, mrows, cp), lambda i: (i, 0, 0)))
    elif shortcut is not None:
        mode = "proj"
        x_sc, spk = shortcut
        cs = x_sc.shape[-1]
        s = jnp.pad(x_sc.astype(jnp.bfloat16),
                    ((0, 0), (0, 0), (0, wp - wo), (0, 0)))
        inputs += [s.reshape(n, mrows, cs), spk["ws"], spk["sc"], spk["b"]]
        in_specs += [
            pl.BlockSpec((nb, mrows, cs), lambda i: (i, 0, 0)),
            pl.BlockSpec(spk["ws"].shape, lambda i: (0, 0)),
            pl.BlockSpec((1, cp), lambda i: (0, 0)),
            pl.BlockSpec((1, cp), lambda i: (0, 0)),
        ]
    else:
        mode = "plain"

    out = pl.pallas_call(
        _conv_body(offs, mrows, relu, mode, nb),
        out_shape=jax.ShapeDtypeStruct((n, mrows, cp), jnp.bfloat16),
        grid=(n // nb,),
        in_specs=in_specs,
        out_specs=pl.BlockSpec((nb, mrows, cp), lambda i: (i, 0, 0)),
        compiler_params=_CP,
    )(*inputs)
    return out.reshape(n, ho, wp, cp)[:, :, :wo, :cout]


def _rdb(x, rp):
    n, h, w, c = x.shape
    wp = w + 2
    mrows = h * wp
    min_rows = (h + 2) * wp + 2
    row0 = wp + 1
    offs = [dy * wp + dx for dy in range(3) for dx in range(3)]
    num_layers = rp["L"]
    nb = 4 if n % 4 == 0 else 1

    xc = _to_canvas(x, 1, 1, 1, 1, min_rows)

    inputs = [xc]
    in_specs = [pl.BlockSpec((nb, min_rows, c), lambda i: (i, 0, 0))]
    for j in range(num_layers):
        wj, bj = rp["ws"][j], rp["bs"][j]
        inputs += [wj, bj]
        in_specs += [pl.BlockSpec(wj.shape, lambda i: (0, 0, 0)),
                     pl.BlockSpec(bj.shape, lambda i: (0, 0))]
    inputs += [rp["wlff"], rp["blff"]]
    in_specs += [pl.BlockSpec(rp["wlff"].shape, lambda i: (0, 0, 0)),
                 pl.BlockSpec(rp["blff"].shape, lambda i: (0, 0))]

    out = pl.pallas_call(
        _rdb_body(num_layers, offs, mrows, row0, wp, w, nb),
        out_shape=jax.ShapeDtypeStruct((n, mrows, c), jnp.bfloat16),
        grid=(n // nb,),
        in_specs=in_specs,
        out_specs=pl.BlockSpec((nb, mrows, c), lambda i: (i, 0, 0)),
        scratch_shapes=[pltpu.VMEM((nb, num_layers + 1, min_rows, c),
                                   jnp.bfloat16)],
        compiler_params=_CP,
    )(*inputs)
    return out.reshape(n, h, wp, c)[:, :, :w, :]


def _space_to_depth(x):
    n, h, w, c = x.shape
    x = x.reshape(n, h // 2, 2, w // 2, 2, c)
    x = jnp.transpose(x, (0, 1, 3, 2, 4, 5))
    return x.reshape(n, h // 2, w // 2, 4 * c)


def _resblock(x, bp):
    if bp["stride"] == 2:
        h = _conv(_space_to_depth(x), bp["c1"], pads=(1, 0, 1, 0), relu=True)
        x_sc = x[:, ::2, ::2, :]
    else:
        h = _conv(x, bp["c1"], relu=True)
        x_sc = x
    if bp["shortcut"] is not None:
        return _conv(h, bp["c2"], relu=True, shortcut=(x_sc, bp["shortcut"]))
    return _conv(h, bp["c2"], relu=True, residual=x_sc)


def _avg_pool_flatten(x, k):
    n, h, w, c = x.shape
    x = x.astype(jnp.float32).reshape(n, h // k, k, w // k, k, c).mean(axis=(2, 4))
    x = jnp.transpose(x, (0, 3, 1, 2))
    return x.reshape(n, -1)


def kernel(x, conv1_wt, conv1_sc, conv1_b, rdb0_dl0_w, rdb0_dl0_b, rdb0_dl1_w, rdb0_dl1_b, rdb0_dl2_w, rdb0_dl2_b, rdb0_wlff, rdb0_blff, rdb1_dl0_w, rdb1_dl0_b, rdb1_dl1_w, rdb1_dl1_b, rdb1_dl2_w, rdb1_dl2_b, rdb1_dl3_w, rdb1_dl3_b, rdb1_wlff, rdb1_blff, gff1_wt, gff1_sc, gff1_b, gff2_wt, gff2_sc, gff2_b, res_layer1_b0_c1_wt, res_layer1_b0_c1_sc, res_layer1_b0_c1_b, res_layer1_b0_c2_wt, res_layer1_b0_c2_sc, res_layer1_b0_c2_b, res_layer1_b1_c1_wt, res_layer1_b1_c1_sc, res_layer1_b1_c1_b, res_layer1_b1_c2_wt, res_layer1_b1_c2_sc, res_layer1_b1_c2_b, res_layer2_b0_c1_wt, res_layer2_b0_c1_sc, res_layer2_b0_c1_b, res_layer2_b0_c2_wt, res_layer2_b0_c2_sc, res_layer2_b0_c2_b, res_layer2_b0_sc_ws, res_layer2_b0_sc_sc, res_layer2_b0_sc_b, res_layer2_b1_c1_wt, res_layer2_b1_c1_sc, res_layer2_b1_c1_b, res_layer2_b1_c2_wt, res_layer2_b1_c2_sc, res_layer2_b1_c2_b, res_layer3_b0_c1_wt, res_layer3_b0_c1_sc, res_layer3_b0_c1_b, res_layer3_b0_c2_wt, res_layer3_b0_c2_sc, res_layer3_b0_c2_b, res_layer3_b0_sc_ws, res_layer3_b0_sc_sc, res_layer3_b0_sc_b, res_layer3_b1_c1_wt, res_layer3_b1_c1_sc, res_layer3_b1_c1_b, res_layer3_b1_c2_wt, res_layer3_b1_c2_sc, res_layer3_b1_c2_b, res_layer4_b0_c1_wt, res_layer4_b0_c1_sc, res_layer4_b0_c1_b, res_layer4_b0_c2_wt, res_layer4_b0_c2_sc, res_layer4_b0_c2_b, res_layer4_b0_sc_ws, res_layer4_b0_sc_sc, res_layer4_b0_sc_b, res_layer4_b1_c1_wt, res_layer4_b1_c1_sc, res_layer4_b1_c1_b, res_layer4_b1_c2_wt, res_layer4_b1_c2_sc, res_layer4_b1_c2_b, res_layer5_b0_c1_wt, res_layer5_b0_c1_sc, res_layer5_b0_c1_b, res_layer5_b0_c2_wt, res_layer5_b0_c2_sc, res_layer5_b0_c2_b, res_layer5_b0_sc_ws, res_layer5_b0_sc_sc, res_layer5_b0_sc_b, res_layer5_b1_c1_wt, res_layer5_b1_c1_sc, res_layer5_b1_c1_b, res_layer5_b1_c2_wt, res_layer5_b1_c2_sc, res_layer5_b1_c2_b, res_fc1_w, res_fc1_b, res_fc2_w, res_fc2_b):
    def _cv(wt, sc, b, cout):
        k = int(round(float(wt.shape[0]) ** 0.5))
        return {"wt": wt, "sc": sc, "b": b, "kh": k, "kw": k,
                "cout": cout, "cp": wt.shape[-1]}

    rdbs = [
        {"L": 3, "ws": [rdb0_dl0_w, rdb0_dl1_w, rdb0_dl2_w],
         "bs": [rdb0_dl0_b, rdb0_dl1_b, rdb0_dl2_b],
         "wlff": rdb0_wlff, "blff": rdb0_blff},
        {"L": 4, "ws": [rdb1_dl0_w, rdb1_dl1_w, rdb1_dl2_w, rdb1_dl3_w],
         "bs": [rdb1_dl0_b, rdb1_dl1_b, rdb1_dl2_b, rdb1_dl3_b],
         "wlff": rdb1_wlff, "blff": rdb1_blff},
    ]
    layers = {
        "layer1": [
            {"stride": 1, "shortcut": None,
             "c1": _cv(res_layer1_b0_c1_wt, res_layer1_b0_c1_sc, res_layer1_b0_c1_b, 64),
             "c2": _cv(res_layer1_b0_c2_wt, res_layer1_b0_c2_sc, res_layer1_b0_c2_b, 64)},
            {"stride": 1, "shortcut": None,
             "c1": _cv(res_layer1_b1_c1_wt, res_layer1_b1_c1_sc, res_layer1_b1_c1_b, 64),
             "c2": _cv(res_layer1_b1_c2_wt, res_layer1_b1_c2_sc, res_layer1_b1_c2_b, 64)},
        ],
        "layer2": [
            {"stride": 2,
             "shortcut": {"ws": res_layer2_b0_sc_ws, "sc": res_layer2_b0_sc_sc, "b": res_layer2_b0_sc_b},
             "c1": _cv(res_layer2_b0_c1_wt, res_layer2_b0_c1_sc, res_layer2_b0_c1_b, 128),
             "c2": _cv(res_layer2_b0_c2_wt, res_layer2_b0_c2_sc, res_layer2_b0_c2_b, 128)},
            {"stride": 1, "shortcut": None,
             "c1": _cv(res_layer2_b1_c1_wt, res_layer2_b1_c1_sc, res_layer2_b1_c1_b, 128),
             "c2": _cv(res_layer2_b1_c2_wt, res_layer2_b1_c2_sc, res_layer2_b1_c2_b, 128)},
        ],
        "layer3": [
            {"stride": 2,
             "shortcut": {"ws": res_layer3_b0_sc_ws, "sc": res_layer3_b0_sc_sc, "b": res_layer3_b0_sc_b},
             "c1": _cv(res_layer3_b0_c1_wt, res_layer3_b0_c1_sc, res_layer3_b0_c1_b, 256),
             "c2": _cv(res_layer3_b0_c2_wt, res_layer3_b0_c2_sc, res_layer3_b0_c2_b, 256)},
            {"stride": 1, "shortcut": None,
             "c1": _cv(res_layer3_b1_c1_wt, res_layer3_b1_c1_sc, res_layer3_b1_c1_b, 256),
             "c2": _cv(res_layer3_b1_c2_wt, res_layer3_b1_c2_sc, res_layer3_b1_c2_b, 256)},
        ],
        "layer4": [
            {"stride": 2,
             "shortcut": {"ws": res_layer4_b0_sc_ws, "sc": res_layer4_b0_sc_sc, "b": res_layer4_b0_sc_b},
             "c1": _cv(res_layer4_b0_c1_wt, res_layer4_b0_c1_sc, res_layer4_b0_c1_b, 512),
             "c2": _cv(res_layer4_b0_c2_wt, res_layer4_b0_c2_sc, res_layer4_b0_c2_b, 512)},
            {"stride": 1, "shortcut": None,
             "c1": _cv(res_layer4_b1_c1_wt, res_layer4_b1_c1_sc, res_layer4_b1_c1_b, 512),
             "c2": _cv(res_layer4_b1_c2_wt, res_layer4_b1_c2_sc, res_layer4_b1_c2_b, 512)},
        ],
        "layer5": [
            {"stride": 2,
             "shortcut": {"ws": res_layer5_b0_sc_ws, "sc": res_layer5_b0_sc_sc, "b": res_layer5_b0_sc_b},
             "c1": _cv(res_layer5_b0_c1_wt, res_layer5_b0_c1_sc, res_layer5_b0_c1_b, 512),
             "c2": _cv(res_layer5_b0_c2_wt, res_layer5_b0_c2_sc, res_layer5_b0_c2_b, 512)},
            {"stride": 1, "shortcut": None,
             "c1": _cv(res_layer5_b1_c1_wt, res_layer5_b1_c1_sc, res_layer5_b1_c1_b, 512),
             "c2": _cv(res_layer5_b1_c2_wt, res_layer5_b1_c2_sc, res_layer5_b1_c2_b, 512)},
        ],
    }

    xh = jnp.transpose(x, (0, 2, 3, 1)).astype(jnp.bfloat16)
    shallow = _conv(xh, _cv(conv1_wt, conv1_sc, conv1_b, 64), relu=False)
    feats = [_rdb(shallow, rp) for rp in rdbs]
    cat = jnp.concatenate(feats, axis=-1)
    g = _conv(cat, _cv(gff1_wt, gff1_sc, gff1_b, 64), relu=False)
    out = _conv(g, _cv(gff2_wt, gff2_sc, gff2_b, 64), relu=False, residual=shallow)

    for name in ("layer1", "layer2", "layer3", "layer4"):
        for bp in layers[name]:
            out = _resblock(out, bp)
    out1 = _avg_pool_flatten(out, 4) @ res_fc1_w.T + res_fc1_b
    out2 = out
    for bp in layers["layer5"]:
        out2 = _resblock(out2, bp)
    out2 = _avg_pool_flatten(out2, 2) @ res_fc2_w.T + res_fc2_b
    return out1, out2


# speedup vs baseline: 1.3041x; 1.3041x over previous
import jax
import jax.numpy as jnp
from jax.experimental import pallas as pl
from jax.experimental.pallas import tpu as pltpu


_CP = pltpu.CompilerParams(
    dimension_semantics=("parallel",),
    vmem_limit_bytes=64 * 1024 * 1024,
)


def _pick_b(n, h, w):
    pix = h * w
    if pix >= 256:
        return 1
    want = 8 if pix >= 64 else 32
    while want > 1 and n % want:
        want //= 2
    return max(want, 1)


def _conv_body(offs, mrows, relu, mode, nb):
    def _accum(x_ref, w_ref):
        acc = None
        for t, off in enumerate(offs):
            xs = x_ref[:, off:off + mrows, :].reshape(nb * mrows, x_ref.shape[-1])
            d = jnp.dot(xs, w_ref[t], preferred_element_type=jnp.float32)
            acc = d if acc is None else acc + d
        return acc

    if mode == "plain":
        def _body(x_ref, w_ref, sc_ref, b_ref, o_ref):
            y = _accum(x_ref, w_ref) * sc_ref[...] + b_ref[...]
            y = jnp.maximum(y, 0.0) if relu else y
            o_ref[...] = y.reshape(o_ref.shape).astype(o_ref.dtype)
    elif mode == "res":
        def _body(x_ref, w_ref, sc_ref, b_ref, r_ref, o_ref):
            y = _accum(x_ref, w_ref) * sc_ref[...] + b_ref[...]
            y = y + r_ref[...].reshape(y.shape).astype(jnp.float32)
            y = jnp.maximum(y, 0.0) if relu else y
            o_ref[...] = y.reshape(o_ref.shape).astype(o_ref.dtype)
    else:
        def _body(x_ref, w_ref, sc_ref, b_ref,
                  s_ref, ws_ref, ssc_ref, sb_ref, o_ref):
            y = _accum(x_ref, w_ref) * sc_ref[...] + b_ref[...]
            ss = s_ref[...].reshape(nb * mrows, s_ref.shape[-1])
            proj = jnp.dot(ss, ws_ref[...], preferred_element_type=jnp.float32)
            y = y + proj * ssc_ref[...] + sb_ref[...]
            y = jnp.maximum(y, 0.0) if relu else y
            o_ref[...] = y.reshape(o_ref.shape).astype(o_ref.dtype)
    return _body


def _rdb_body(num_layers, offs, mrows, row0, wpad, wvalid, nb):
    ntaps = len(offs)

    def _body(*refs):
        x_ref = refs[0]
        lw = refs[1:1 + 2 * num_layers]
        wlff_ref = refs[1 + 2 * num_layers]
        blff_ref = refs[2 + 2 * num_layers]
        o_ref = refs[3 + 2 * num_layers]
        canvas = refs[4 + 2 * num_layers]

        canvas[...] = jnp.zeros_like(canvas)
        canvas[:, 0] = x_ref[...]

        col = jax.lax.broadcasted_iota(jnp.int32, (nb * mrows, 1), 0) % wpad
        valid = col < wvalid

        for j in range(num_layers):
            w_ref, b_ref = lw[2 * j], lw[2 * j + 1]
            acc = None
            for c in range(j + 1):
                for t, off in enumerate(offs):
                    xs = canvas[:, c, off:off + mrows, :].reshape(nb * mrows, 64)
                    d = jnp.dot(xs, w_ref[c * ntaps + t],
                                preferred_element_type=jnp.float32)
                    acc = d if acc is None else acc + d
            y = jnp.maximum(acc + b_ref[...], 0.0)
            y = jnp.where(valid, y, 0.0)
            canvas[:, j + 1, row0:row0 + mrows, :] = (
                y.reshape(nb, mrows, 64).astype(canvas.dtype))

        acc = None
        for c in range(num_layers + 1):
            xs = canvas[:, c, row0:row0 + mrows, :].reshape(nb * mrows, 64)
            d = jnp.dot(xs, wlff_ref[c], preferred_element_type=jnp.float32)
            acc = d if acc is None else acc + d
        res = canvas[:, 0, row0:row0 + mrows, :].reshape(nb * mrows, 64)
        out = acc + blff_ref[...] + res.astype(jnp.float32)
        o_ref[...] = out.reshape(nb, mrows, 64).astype(o_ref.dtype)

    return _body


def _to_canvas(x, pt, pb, pleft, pr, min_rows):
    n, h, w, c = x.shape
    xc = jnp.pad(x.astype(jnp.bfloat16), ((0, 0), (pt, pb), (pleft, pr), (0, 0)))
    hp, wp = h + pt + pb, w + pleft + pr
    xc = xc.reshape(n, hp * wp, c)
    if min_rows > hp * wp:
        xc = jnp.pad(xc, ((0, 0), (0, min_rows - hp * wp), (0, 0)))
    return xc


def _conv(x, pk, *, pads=None, relu=False, residual=None, shortcut=None):
    n, h, w, cin = x.shape
    kh, kw, cout, cp = pk["kh"], pk["kw"], pk["cout"], pk["cp"]
    if pads is None:
        pads = (kh // 2, kh // 2, kw // 2, kw // 2)
    pt, pb, pleft, pr = pads
    hp, wp = h + pt + pb, w + pleft + pr
    ho, wo = hp - kh + 1, wp - kw + 1
    mrows = ho * wp
    min_rows = hp * wp + (kw - 1)
    offs = [dy * wp + dx for dy in range(kh) for dx in range(kw)]
    nb = _pick_b(n, h, w)

    xc = _to_canvas(x, pt, pb, pleft, pr, min_rows)

    inputs = [xc, pk["wt"], pk["sc"], pk["b"]]
    in_specs = [
        pl.BlockSpec((nb, min_rows, cin), lambda i: (i, 0, 0)),
        pl.BlockSpec(pk["wt"].shape, lambda i: (0, 0, 0)),
        pl.BlockSpec((1, cp), lambda i: (0, 0)),
        pl.BlockSpec((1, cp), lambda i: (0, 0)),
    ]

    if residual is not None:
        mode = "res"
        r = jnp.pad(residual.astype(jnp.bfloat16),
                    ((0, 0), (0, 0), (0, wp - wo), (0, cp - cout)))
        inputs.append(r.reshape(n, mrows, cp))
        in_specs.append(pl.BlockSpec((nb, mrows, cp), lambda i: (i, 0, 0)))
    elif shortcut is not None:
        mode = "proj"
        x_sc, spk = shortcut
        cs = x_sc.shape[-1]
        s = jnp.pad(x_sc.astype(jnp.bfloat16),
                    ((0, 0), (0, 0), (0, wp - wo), (0, 0)))
        inputs += [s.reshape(n, mrows, cs), spk["ws"], spk["sc"], spk["b"]]
        in_specs += [
            pl.BlockSpec((nb, mrows, cs), lambda i: (i, 0, 0)),
            pl.BlockSpec(spk["ws"].shape, lambda i: (0, 0)),
            pl.BlockSpec((1, cp), lambda i: (0, 0)),
            pl.BlockSpec((1, cp), lambda i: (0, 0)),
        ]
    else:
        mode = "plain"

    out = pl.pallas_call(
        _conv_body(offs, mrows, relu, mode, nb),
        out_shape=jax.ShapeDtypeStruct((n, mrows, cp), jnp.bfloat16),
        grid=(n // nb,),
        in_specs=in_specs,
        out_specs=pl.BlockSpec((nb, mrows, cp), lambda i: (i, 0, 0)),
        compiler_params=_CP,
    )(*inputs)
    return out.reshape(n, ho, wp, cp)[:, :, :wo, :cout]


def _rdb(x, rp):
    n, h, w, c = x.shape
    wp = w + 2
    mrows = h * wp
    min_rows = (h + 2) * wp + 2
    row0 = wp + 1
    offs = [dy * wp + dx for dy in range(3) for dx in range(3)]
    num_layers = rp["L"]
    nb = 1

    xc = _to_canvas(x, 1, 1, 1, 1, min_rows)

    inputs = [xc]
    in_specs = [pl.BlockSpec((nb, min_rows, c), lambda i: (i, 0, 0))]
    for j in range(num_layers):
        wj, bj = rp["ws"][j], rp["bs"][j]
        inputs += [wj, bj]
        in_specs += [pl.BlockSpec(wj.shape, lambda i: (0, 0, 0)),
                     pl.BlockSpec(bj.shape, lambda i: (0, 0))]
    inputs += [rp["wlff"], rp["blff"]]
    in_specs += [pl.BlockSpec(rp["wlff"].shape, lambda i: (0, 0, 0)),
                 pl.BlockSpec(rp["blff"].shape, lambda i: (0, 0))]

    out = pl.pallas_call(
        _rdb_body(num_layers, offs, mrows, row0, wp, w, nb),
        out_shape=jax.ShapeDtypeStruct((n, mrows, c), jnp.bfloat16),
        grid=(n // nb,),
        in_specs=in_specs,
        out_specs=pl.BlockSpec((nb, mrows, c), lambda i: (i, 0, 0)),
        scratch_shapes=[pltpu.VMEM((nb, num_layers + 1, min_rows, c),
                                   jnp.bfloat16)],
        compiler_params=_CP,
    )(*inputs)
    return out.reshape(n, h, wp, c)[:, :, :w, :]


def _space_to_depth(x):
    n, h, w, c = x.shape
    x = x.reshape(n, h // 2, 2, w // 2, 2, c)
    x = jnp.transpose(x, (0, 1, 3, 2, 4, 5))
    return x.reshape(n, h // 2, w // 2, 4 * c)


def _resblock(x, bp):
    if bp["stride"] == 2:
        h = _conv(_space_to_depth(x), bp["c1"], pads=(1, 0, 1, 0), relu=True)
        x_sc = x[:, ::2, ::2, :]
    else:
        h = _conv(x, bp["c1"], relu=True)
        x_sc = x
    if bp["shortcut"] is not None:
        return _conv(h, bp["c2"], relu=True, shortcut=(x_sc, bp["shortcut"]))
    return _conv(h, bp["c2"], relu=True, residual=x_sc)


def _avg_pool_flatten(x, k):
    n, h, w, c = x.shape
    x = x.astype(jnp.float32).reshape(n, h // k, k, w // k, k, c).mean(axis=(2, 4))
    x = jnp.transpose(x, (0, 3, 1, 2))
    return x.reshape(n, -1)


def kernel(x, conv1_wt, conv1_sc, conv1_b, rdb0_dl0_w, rdb0_dl0_b, rdb0_dl1_w, rdb0_dl1_b, rdb0_dl2_w, rdb0_dl2_b, rdb0_wlff, rdb0_blff, rdb1_dl0_w, rdb1_dl0_b, rdb1_dl1_w, rdb1_dl1_b, rdb1_dl2_w, rdb1_dl2_b, rdb1_dl3_w, rdb1_dl3_b, rdb1_wlff, rdb1_blff, gff1_wt, gff1_sc, gff1_b, gff2_wt, gff2_sc, gff2_b, res_layer1_b0_c1_wt, res_layer1_b0_c1_sc, res_layer1_b0_c1_b, res_layer1_b0_c2_wt, res_layer1_b0_c2_sc, res_layer1_b0_c2_b, res_layer1_b1_c1_wt, res_layer1_b1_c1_sc, res_layer1_b1_c1_b, res_layer1_b1_c2_wt, res_layer1_b1_c2_sc, res_layer1_b1_c2_b, res_layer2_b0_c1_wt, res_layer2_b0_c1_sc, res_layer2_b0_c1_b, res_layer2_b0_c2_wt, res_layer2_b0_c2_sc, res_layer2_b0_c2_b, res_layer2_b0_sc_ws, res_layer2_b0_sc_sc, res_layer2_b0_sc_b, res_layer2_b1_c1_wt, res_layer2_b1_c1_sc, res_layer2_b1_c1_b, res_layer2_b1_c2_wt, res_layer2_b1_c2_sc, res_layer2_b1_c2_b, res_layer3_b0_c1_wt, res_layer3_b0_c1_sc, res_layer3_b0_c1_b, res_layer3_b0_c2_wt, res_layer3_b0_c2_sc, res_layer3_b0_c2_b, res_layer3_b0_sc_ws, res_layer3_b0_sc_sc, res_layer3_b0_sc_b, res_layer3_b1_c1_wt, res_layer3_b1_c1_sc, res_layer3_b1_c1_b, res_layer3_b1_c2_wt, res_layer3_b1_c2_sc, res_layer3_b1_c2_b, res_layer4_b0_c1_wt, res_layer4_b0_c1_sc, res_layer4_b0_c1_b, res_layer4_b0_c2_wt, res_layer4_b0_c2_sc, res_layer4_b0_c2_b, res_layer4_b0_sc_ws, res_layer4_b0_sc_sc, res_layer4_b0_sc_b, res_layer4_b1_c1_wt, res_layer4_b1_c1_sc, res_layer4_b1_c1_b, res_layer4_b1_c2_wt, res_layer4_b1_c2_sc, res_layer4_b1_c2_b, res_layer5_b0_c1_wt, res_layer5_b0_c1_sc, res_layer5_b0_c1_b, res_layer5_b0_c2_wt, res_layer5_b0_c2_sc, res_layer5_b0_c2_b, res_layer5_b0_sc_ws, res_layer5_b0_sc_sc, res_layer5_b0_sc_b, res_layer5_b1_c1_wt, res_layer5_b1_c1_sc, res_layer5_b1_c1_b, res_layer5_b1_c2_wt, res_layer5_b1_c2_sc, res_layer5_b1_c2_b, res_fc1_w, res_fc1_b, res_fc2_w, res_fc2_b):
    def _cv(wt, sc, b, cout):
        k = int(round(float(wt.shape[0]) ** 0.5))
        return {"wt": wt, "sc": sc, "b": b, "kh": k, "kw": k,
                "cout": cout, "cp": wt.shape[-1]}

    rdbs = [
        {"L": 3, "ws": [rdb0_dl0_w, rdb0_dl1_w, rdb0_dl2_w],
         "bs": [rdb0_dl0_b, rdb0_dl1_b, rdb0_dl2_b],
         "wlff": rdb0_wlff, "blff": rdb0_blff},
        {"L": 4, "ws": [rdb1_dl0_w, rdb1_dl1_w, rdb1_dl2_w, rdb1_dl3_w],
         "bs": [rdb1_dl0_b, rdb1_dl1_b, rdb1_dl2_b, rdb1_dl3_b],
         "wlff": rdb1_wlff, "blff": rdb1_blff},
    ]
    layers = {
        "layer1": [
            {"stride": 1, "shortcut": None,
             "c1": _cv(res_layer1_b0_c1_wt, res_layer1_b0_c1_sc, res_layer1_b0_c1_b, 64),
             "c2": _cv(res_layer1_b0_c2_wt, res_layer1_b0_c2_sc, res_layer1_b0_c2_b, 64)},
            {"stride": 1, "shortcut": None,
             "c1": _cv(res_layer1_b1_c1_wt, res_layer1_b1_c1_sc, res_layer1_b1_c1_b, 64),
             "c2": _cv(res_layer1_b1_c2_wt, res_layer1_b1_c2_sc, res_layer1_b1_c2_b, 64)},
        ],
        "layer2": [
            {"stride": 2,
             "shortcut": {"ws": res_layer2_b0_sc_ws, "sc": res_layer2_b0_sc_sc, "b": res_layer2_b0_sc_b},
             "c1": _cv(res_layer2_b0_c1_wt, res_layer2_b0_c1_sc, res_layer2_b0_c1_b, 128),
             "c2": _cv(res_layer2_b0_c2_wt, res_layer2_b0_c2_sc, res_layer2_b0_c2_b, 128)},
            {"stride": 1, "shortcut": None,
             "c1": _cv(res_layer2_b1_c1_wt, res_layer2_b1_c1_sc, res_layer2_b1_c1_b, 128),
             "c2": _cv(res_layer2_b1_c2_wt, res_layer2_b1_c2_sc, res_layer2_b1_c2_b, 128)},
        ],
        "layer3": [
            {"stride": 2,
             "shortcut": {"ws": res_layer3_b0_sc_ws, "sc": res_layer3_b0_sc_sc, "b": res_layer3_b0_sc_b},
             "c1": _cv(res_layer3_b0_c1_wt, res_layer3_b0_c1_sc, res_layer3_b0_c1_b, 256),
             "c2": _cv(res_layer3_b0_c2_wt, res_layer3_b0_c2_sc, res_layer3_b0_c2_b, 256)},
            {"stride": 1, "shortcut": None,
             "c1": _cv(res_layer3_b1_c1_wt, res_layer3_b1_c1_sc, res_layer3_b1_c1_b, 256),
             "c2": _cv(res_layer3_b1_c2_wt, res_layer3_b1_c2_sc, res_layer3_b1_c2_b, 256)},
        ],
        "layer4": [
            {"stride": 2,
             "shortcut": {"ws": res_layer4_b0_sc_ws, "sc": res_layer4_b0_sc_sc, "b": res_layer4_b0_sc_b},
             "c1": _cv(res_layer4_b0_c1_wt, res_layer4_b0_c1_sc, res_layer4_b0_c1_b, 512),
             "c2": _cv(res_layer4_b0_c2_wt, res_layer4_b0_c2_sc, res_layer4_b0_c2_b, 512)},
            {"stride": 1, "shortcut": None,
             "c1": _cv(res_layer4_b1_c1_wt, res_layer4_b1_c1_sc, res_layer4_b1_c1_b, 512),
             "c2": _cv(res_layer4_b1_c2_wt, res_layer4_b1_c2_sc, res_layer4_b1_c2_b, 512)},
        ],
        "layer5": [
            {"stride": 2,
             "shortcut": {"ws": res_layer5_b0_sc_ws, "sc": res_layer5_b0_sc_sc, "b": res_layer5_b0_sc_b},
             "c1": _cv(res_layer5_b0_c1_wt, res_layer5_b0_c1_sc, res_layer5_b0_c1_b, 512),
             "c2": _cv(res_layer5_b0_c2_wt, res_layer5_b0_c2_sc, res_layer5_b0_c2_b, 512)},
            {"stride": 1, "shortcut": None,
             "c1": _cv(res_layer5_b1_c1_wt, res_layer5_b1_c1_sc, res_layer5_b1_c1_b, 512),
             "c2": _cv(res_layer5_b1_c2_wt, res_layer5_b1_c2_sc, res_layer5_b1_c2_b, 512)},
        ],
    }

    xh = jnp.transpose(x, (0, 2, 3, 1)).astype(jnp.bfloat16)
    shallow = _conv(xh, _cv(conv1_wt, conv1_sc, conv1_b, 64), relu=False)
    feats = [_rdb(shallow, rp) for rp in rdbs]
    cat = jnp.concatenate(feats, axis=-1)
    g = _conv(cat, _cv(gff1_wt, gff1_sc, gff1_b, 64), relu=False)
    out = _conv(g, _cv(gff2_wt, gff2_sc, gff2_b, 64), relu=False, residual=shallow)

    for name in ("layer1", "layer2", "layer3", "layer4"):
        for bp in layers[name]:
            out = _resblock(out, bp)
    out1 = _avg_pool_flatten(out, 4) @ res_fc1_w.T + res_fc1_b
    out2 = out
    for bp in layers["layer5"]:
        out2 = _resblock(out2, bp)
    out2 = _avg_pool_flatten(out2, 2) @ res_fc2_w.T + res_fc2_b
    return out1, out2
